# baseline (speedup 1.0000x reference)
import functools

import jax
import jax.numpy as jnp
from jax import lax
from jax.experimental import pallas as pl
from jax.experimental.pallas import tpu as pltpu
from jax.experimental.pallas import tpu_sc as plsc

_NC = 2
_NS = 16
_L = 16


def _sc_body(x_hbm, scale_hbm, out_hbm, xbuf, sbuf, tbuf, *, k, rows,
             row_len, chunk):
    cid = lax.axis_index("c")
    sid = lax.axis_index("s")
    wid = sid * _NC + cid
    rows_w = rows // (_NC * _NS)
    base_row = wid * rows_w
    vpr = row_len // _L

    pltpu.sync_copy(scale_hbm, sbuf)
    alpha = sbuf[pl.ds(0, _L)]
    beta = sbuf[pl.ds(_L, _L)]

    zero_i = jnp.zeros((_L,), jnp.int32)
    zero_f = jnp.zeros((_L,), jnp.float32)
    one_f = jnp.ones((_L,), jnp.float32)
    kvf = jnp.full((_L,), float(k), jnp.float32)
    kf = jnp.float32(k)
    sign_off = jnp.full((_L,), 0x7FFFFFFF, jnp.int32)
    t_one = jnp.full((_L,), 0x3F800000, jnp.int32)
    t_two = jnp.full((_L,), 0x40000000, jnp.int32)
    bit22 = jnp.full((_L,), 1 << 22, jnp.int32)
    bit29 = jnp.full((_L,), 1 << 29, jnp.int32)
    lane = lax.iota(jnp.int32, _L)
    bf1 = lane ^ 1
    bf2 = lane ^ 2
    bf4 = lane ^ 4
    bf8 = lane ^ 8

    def lanesum(acc):
        s = acc + acc.at[bf1].get(mode="promise_in_bounds")
        s = s + s.at[bf2].get(mode="promise_in_bounds")
        s = s + s.at[bf4].get(mode="promise_in_bounds")
        return s + s.at[bf8].get(mode="promise_in_bounds")

    def chunk_body(g, _):
        row0 = base_row + g * chunk
        pltpu.sync_copy(x_hbm.at[pl.ds(row0, chunk)], xbuf)

        def row_body(r, _):
            def count_ge(trial):
                def sb(v, acc):
                    xv = xbuf[r, pl.ds(v * _L, _L)]
                    av = lax.bitcast_convert_type(xv, jnp.int32) & sign_off
                    return acc + jnp.where(av >= trial, one_f, zero_f)

                acc = lax.fori_loop(0, vpr, sb, zero_f, unroll=16)
                return lanesum(acc)

            def ge_mask(cnt):
                ge = jnp.minimum(jnp.maximum(cnt - kvf + one_f, zero_f),
                                 one_f).astype(jnp.int32)
                return zero_i - ge

            c1 = count_ge(t_one)
            c2 = count_ge(t_two)
            n1 = ge_mask(c1)
            n2 = ge_mask(c2)
            inr = n1 & ~n2
            t0 = (t_one & inr) | (t_two & n2)
            bitv0 = (bit22 & inr) | (bit29 & ~inr)
            tbuf[pl.ds(0, _L)] = lax.bitcast_convert_type(t0, jnp.float32)
            done0 = jnp.logical_or(c1[0] == kf, c2[0] == kf)

            def wb(i, st):
                done, bitv = st

                def live(_):
                    tf = tbuf[pl.ds(0, _L)]
                    t = lax.bitcast_convert_type(tf, jnp.int32)
                    trial = t | bitv
                    cnt = count_ge(trial)
                    nt = t | (bitv & ge_mask(cnt))
                    tbuf[pl.ds(0, _L)] = lax.bitcast_convert_type(
                        nt, jnp.float32)
                    return cnt[0] == kf

                done = lax.cond(done, lambda _: done, live, 0)
                return (jnp.logical_or(done, bitv[0] == 1), bitv >> 1)

            lax.fori_loop(0, 30, wb, (done0, bitv0))
            tv = lax.bitcast_convert_type(tbuf[pl.ds(0, _L)], jnp.int32)

            def mb(v, _):
                xv = xbuf[r, pl.ds(v * _L, _L)]
                av = lax.bitcast_convert_type(xv, jnp.int32) & sign_off
                sel = jnp.where(av >= tv, xv, zero_f)
                xbuf[r, pl.ds(v * _L, _L)] = sel * alpha + xv * beta
                return 0

            lax.fori_loop(0, vpr, mb, 0, unroll=16)
            return 0

        lax.fori_loop(0, chunk, row_body, 0)
        pltpu.sync_copy(xbuf, out_hbm.at[pl.ds(row0, chunk)])
        return 0

    lax.fori_loop(0, rows_w // chunk, chunk_body, 0)


def _tc_body(scale_ref, x_ref, o_ref, *, k: int):
    xb = x_ref[...]
    a = jax.lax.bitcast_convert_type(jnp.abs(xb), jnp.int32)
    r = xb.shape[0]
    t = jnp.zeros((r, 1), jnp.int32)
    for b in range(30, -1, -1):
        trial = t + (1 << b)
        cnt = jnp.sum((a >= trial).astype(jnp.int32), axis=1, keepdims=True)
        t = jnp.where(cnt >= k, trial, t)
    sparse = jnp.where(a >= t, xb, jnp.zeros_like(xb))
    o_ref[...] = sparse * scale_ref[0] + xb * scale_ref[1]


def _sc_call(x2, scale, *, k, sc_rows, hw, chunk):
    rows_total = x2.shape[0]
    mesh = plsc.VectorSubcoreMesh(core_axis_name="c", subcore_axis_name="s")
    f = functools.partial(
        pl.kernel,
        mesh=mesh,
        out_type=jax.ShapeDtypeStruct((rows_total, hw), x2.dtype),
        scratch_types=[
            pltpu.VMEM((chunk, hw), jnp.float32),
            pltpu.VMEM((2 * _L,), jnp.float32),
            pltpu.VMEM((_L,), jnp.float32),
        ],
    )(functools.partial(_sc_body, k=k, rows=sc_rows, row_len=hw,
                        chunk=chunk))
    return f(x2, scale)


def _tc_call(x2, scale, *, k, hw, row_off):
    rows = x2.shape[0] - row_off
    blk = 256
    while rows % blk or (row_off and row_off % blk):
        blk //= 2
    off = row_off // blk if blk else 0
    return pl.pallas_call(
        functools.partial(_tc_body, k=k),
        grid=(rows // blk,),
        in_specs=[
            pl.BlockSpec(memory_space=pltpu.SMEM),
            pl.BlockSpec((blk, hw), lambda i: (i + off, 0)),
        ],
        out_specs=pl.BlockSpec((blk, hw), lambda i: (i + off, 0)),
        out_shape=jax.ShapeDtypeStruct((x2.shape[0], hw), x2.dtype),
    )(scale, x2)


def kernel(x, tau):
    n, c, h, w = x.shape
    hw = h * w
    k = max(int(0.1 * hw), 1)
    rows = n * c
    x2 = x.reshape(rows, hw)

    tau_f = jnp.asarray(tau, x.dtype)
    is_id = tau_f == jnp.asarray(1.0, x.dtype)
    alpha = jnp.where(is_id, jnp.asarray(1.0, x.dtype), tau_f)
    beta = jnp.where(is_id, jnp.asarray(0.0, x.dtype), 1.0 - tau_f)
    scale_sc = jnp.concatenate(
        [jnp.full((_L,), alpha, x.dtype), jnp.full((_L,), beta, x.dtype)])
    scale_tc = jnp.stack([alpha, beta])

    chunk = 16
    grain = _NC * _NS * chunk
    sc_rows = ((rows * 3) // 8) // grain * grain
    if sc_rows == 0 or rows % grain:
        sc_rows = rows - rows % grain
    tc_rows = rows - sc_rows

    if sc_rows and tc_rows:
        sc_out = _sc_call(x2[:sc_rows], scale_sc, k=k, sc_rows=sc_rows,
                          hw=hw, chunk=chunk)
        tc_out = _tc_call(x2[sc_rows:], scale_tc, k=k, hw=hw, row_off=0)
        out = jnp.concatenate([sc_out, tc_out], axis=0)
    elif sc_rows:
        out = _sc_call(x2, scale_sc, k=k, sc_rows=sc_rows, hw=hw,
                       chunk=chunk)
    else:
        out = _tc_call(x2, scale_tc, k=k, hw=hw, row_off=0)
    return out.reshape(n, c, h, w)


# speedup vs baseline: 1.1365x; 1.1365x over previous
import functools

import jax
import jax.numpy as jnp
from jax import lax
from jax.experimental import pallas as pl
from jax.experimental.pallas import tpu as pltpu
from jax.experimental.pallas import tpu_sc as plsc

_NC = 2
_NS = 16
_L = 16


def _sc_body(x_hbm, scale_hbm, out_hbm, xbuf, sbuf, tbuf, *, k, rows,
             row_len, chunk):
    cid = lax.axis_index("c")
    sid = lax.axis_index("s")
    wid = sid * _NC + cid
    rows_w = rows // (_NC * _NS)
    base_row = wid * rows_w
    vpr = row_len // _L

    pltpu.sync_copy(scale_hbm, sbuf)
    alpha = sbuf[pl.ds(0, _L)]
    beta = sbuf[pl.ds(_L, _L)]

    zero_i = jnp.zeros((_L,), jnp.int32)
    zero_f = jnp.zeros((_L,), jnp.float32)
    one_f = jnp.ones((_L,), jnp.float32)
    kvf = jnp.full((_L,), float(k), jnp.float32)
    kf = jnp.float32(k)
    sign_off = jnp.full((_L,), 0x7FFFFFFF, jnp.int32)
    t_one = jnp.full((_L,), 0x3F800000, jnp.int32)
    t_two = jnp.full((_L,), 0x40000000, jnp.int32)
    bit22 = jnp.full((_L,), 1 << 22, jnp.int32)
    bit29 = jnp.full((_L,), 1 << 29, jnp.int32)
    lane = lax.iota(jnp.int32, _L)
    bf1 = lane ^ 1
    bf2 = lane ^ 2
    bf4 = lane ^ 4
    bf8 = lane ^ 8

    def lanesum(acc):
        s = acc + acc.at[bf1].get(mode="promise_in_bounds")
        s = s + s.at[bf2].get(mode="promise_in_bounds")
        s = s + s.at[bf4].get(mode="promise_in_bounds")
        return s + s.at[bf8].get(mode="promise_in_bounds")

    def chunk_body(g, _):
        row0 = base_row + g * chunk
        pltpu.sync_copy(x_hbm.at[pl.ds(row0, chunk)], xbuf)

        def row_body(r, _):
            def count_ge(trial):
                def sb(v, acc):
                    xv = xbuf[r, pl.ds(v * _L, _L)]
                    av = lax.bitcast_convert_type(xv, jnp.int32) & sign_off
                    return acc + jnp.where(av >= trial, one_f, zero_f)

                acc = lax.fori_loop(0, vpr, sb, zero_f, unroll=16)
                return lanesum(acc)

            def ge_mask(cnt):
                ge = jnp.minimum(jnp.maximum(cnt - kvf + one_f, zero_f),
                                 one_f).astype(jnp.int32)
                return zero_i - ge

            c1 = count_ge(t_one)
            c2 = count_ge(t_two)
            n1 = ge_mask(c1)
            n2 = ge_mask(c2)
            inr = n1 & ~n2
            t0 = (t_one & inr) | (t_two & n2)
            bitv0 = (bit22 & inr) | (bit29 & ~inr)
            tbuf[pl.ds(0, _L)] = lax.bitcast_convert_type(t0, jnp.float32)
            done0 = jnp.logical_or(c1[0] == kf, c2[0] == kf)

            def wb(i, st):
                done, bitv = st

                def live(_):
                    tf = tbuf[pl.ds(0, _L)]
                    t = lax.bitcast_convert_type(tf, jnp.int32)
                    trial = t | bitv
                    cnt = count_ge(trial)
                    nt = t | (bitv & ge_mask(cnt))
                    tbuf[pl.ds(0, _L)] = lax.bitcast_convert_type(
                        nt, jnp.float32)
                    return cnt[0] == kf

                done = lax.cond(done, lambda _: done, live, 0)
                return (jnp.logical_or(done, bitv[0] == 1), bitv >> 1)

            lax.fori_loop(0, 30, wb, (done0, bitv0))
            tv = lax.bitcast_convert_type(tbuf[pl.ds(0, _L)], jnp.int32)

            def mb(v, _):
                xv = xbuf[r, pl.ds(v * _L, _L)]
                av = lax.bitcast_convert_type(xv, jnp.int32) & sign_off
                sel = jnp.where(av >= tv, xv, zero_f)
                xbuf[r, pl.ds(v * _L, _L)] = sel * alpha + xv * beta
                return 0

            lax.fori_loop(0, vpr, mb, 0, unroll=16)
            return 0

        lax.fori_loop(0, chunk, row_body, 0)
        pltpu.sync_copy(xbuf, out_hbm.at[pl.ds(row0, chunk)])
        return 0

    lax.fori_loop(0, rows_w // chunk, chunk_body, 0)


def _tc_body(scale_ref, x_ref, o_ref, *, k: int):
    xb = x_ref[...]
    a = jax.lax.bitcast_convert_type(jnp.abs(xb), jnp.int32)
    r = xb.shape[0]
    t = jnp.zeros((r, 1), jnp.int32)
    for b in range(30, -1, -1):
        trial = t + (1 << b)
        cnt = jnp.sum((a >= trial).astype(jnp.int32), axis=1, keepdims=True)
        t = jnp.where(cnt >= k, trial, t)
    sparse = jnp.where(a >= t, xb, jnp.zeros_like(xb))
    o_ref[...] = sparse * scale_ref[0] + xb * scale_ref[1]


def _sc_call(x2, scale, *, k, sc_rows, hw, chunk):
    rows_total = x2.shape[0]
    mesh = plsc.VectorSubcoreMesh(core_axis_name="c", subcore_axis_name="s")
    f = functools.partial(
        pl.kernel,
        mesh=mesh,
        out_type=jax.ShapeDtypeStruct((rows_total, hw), x2.dtype),
        scratch_types=[
            pltpu.VMEM((chunk, hw), jnp.float32),
            pltpu.VMEM((2 * _L,), jnp.float32),
            pltpu.VMEM((_L,), jnp.float32),
        ],
    )(functools.partial(_sc_body, k=k, rows=sc_rows, row_len=hw,
                        chunk=chunk))
    return f(x2, scale)


def _tc_call(x2, scale, *, k, hw, row_off):
    rows = x2.shape[0] - row_off
    blk = 512
    while rows % blk or (row_off and row_off % blk):
        blk //= 2
    off = row_off // blk if blk else 0
    return pl.pallas_call(
        functools.partial(_tc_body, k=k),
        grid=(rows // blk,),
        in_specs=[
            pl.BlockSpec(memory_space=pltpu.SMEM),
            pl.BlockSpec((blk, hw), lambda i: (i + off, 0)),
        ],
        out_specs=pl.BlockSpec((blk, hw), lambda i: (i + off, 0)),
        out_shape=jax.ShapeDtypeStruct((x2.shape[0], hw), x2.dtype),
    )(scale, x2)


def kernel(x, tau):
    n, c, h, w = x.shape
    hw = h * w
    k = max(int(0.1 * hw), 1)
    rows = n * c
    x2 = x.reshape(rows, hw)

    tau_f = jnp.asarray(tau, x.dtype)
    is_id = tau_f == jnp.asarray(1.0, x.dtype)
    alpha = jnp.where(is_id, jnp.asarray(1.0, x.dtype), tau_f)
    beta = jnp.where(is_id, jnp.asarray(0.0, x.dtype), 1.0 - tau_f)
    scale_sc = jnp.concatenate(
        [jnp.full((_L,), alpha, x.dtype), jnp.full((_L,), beta, x.dtype)])
    scale_tc = jnp.stack([alpha, beta])

    chunk = 16
    grain = _NC * _NS * chunk
    sc_rows = ((rows * 1) // 3) // grain * grain
    if sc_rows == 0 or rows % grain:
        sc_rows = rows - rows % grain
    tc_rows = rows - sc_rows

    if sc_rows and tc_rows:
        sc_out = _sc_call(x2[:sc_rows], scale_sc, k=k, sc_rows=sc_rows,
                          hw=hw, chunk=chunk)
        tc_out = _tc_call(x2[sc_rows:], scale_tc, k=k, hw=hw, row_off=0)
        out = jnp.concatenate([sc_out, tc_out], axis=0)
    elif sc_rows:
        out = _sc_call(x2, scale_sc, k=k, sc_rows=sc_rows, hw=hw,
                       chunk=chunk)
    else:
        out = _tc_call(x2, scale_tc, k=k, hw=hw, row_off=0)
    return out.reshape(n, c, h, w)


# speedup vs baseline: 1.1381x; 1.0013x over previous
import functools

import jax
import jax.numpy as jnp
from jax import lax
from jax.experimental import pallas as pl
from jax.experimental.pallas import tpu as pltpu
from jax.experimental.pallas import tpu_sc as plsc

_NC = 2
_NS = 16
_L = 16


def _sc_body(x_hbm, scale_hbm, out_hbm, xbuf, sbuf, tbuf, *, k, rows,
             row_len, chunk):
    cid = lax.axis_index("c")
    sid = lax.axis_index("s")
    wid = sid * _NC + cid
    rows_w = rows // (_NC * _NS)
    base_row = wid * rows_w
    vpr = row_len // _L

    pltpu.sync_copy(scale_hbm, sbuf)
    alpha = sbuf[pl.ds(0, _L)]
    beta = sbuf[pl.ds(_L, _L)]

    zero_i = jnp.zeros((_L,), jnp.int32)
    zero_f = jnp.zeros((_L,), jnp.float32)
    one_f = jnp.ones((_L,), jnp.float32)
    kvf = jnp.full((_L,), float(k), jnp.float32)
    kf = jnp.float32(k)
    sign_off = jnp.full((_L,), 0x7FFFFFFF, jnp.int32)
    t_one = jnp.full((_L,), 0x3F800000, jnp.int32)
    t_two = jnp.full((_L,), 0x40000000, jnp.int32)
    bit22 = jnp.full((_L,), 1 << 22, jnp.int32)
    bit29 = jnp.full((_L,), 1 << 29, jnp.int32)
    lane = lax.iota(jnp.int32, _L)
    bf1 = lane ^ 1
    bf2 = lane ^ 2
    bf4 = lane ^ 4
    bf8 = lane ^ 8

    def lanesum(acc):
        s = acc + acc.at[bf1].get(mode="promise_in_bounds")
        s = s + s.at[bf2].get(mode="promise_in_bounds")
        s = s + s.at[bf4].get(mode="promise_in_bounds")
        return s + s.at[bf8].get(mode="promise_in_bounds")

    def chunk_body(g, _):
        row0 = base_row + g * chunk
        pltpu.sync_copy(x_hbm.at[pl.ds(row0, chunk)], xbuf)

        def row_body(r, _):
            def count_ge(trial):
                def sb(v, acc):
                    xv = xbuf[r, pl.ds(v * _L, _L)]
                    av = lax.bitcast_convert_type(xv, jnp.int32) & sign_off
                    return acc + jnp.where(av >= trial, one_f, zero_f)

                acc = lax.fori_loop(0, vpr, sb, zero_f, unroll=16)
                return lanesum(acc)

            def ge_mask(cnt):
                ge = jnp.minimum(jnp.maximum(cnt - kvf + one_f, zero_f),
                                 one_f).astype(jnp.int32)
                return zero_i - ge

            c1 = count_ge(t_one)
            c2 = count_ge(t_two)
            n1 = ge_mask(c1)
            n2 = ge_mask(c2)
            inr = n1 & ~n2
            t0 = (t_one & inr) | (t_two & n2)
            bitv0 = (bit22 & inr) | (bit29 & ~inr)
            tbuf[pl.ds(0, _L)] = lax.bitcast_convert_type(t0, jnp.float32)
            done0 = jnp.logical_or(c1[0] == kf, c2[0] == kf)

            def wb(i, st):
                done, bitv = st

                def live(_):
                    tf = tbuf[pl.ds(0, _L)]
                    t = lax.bitcast_convert_type(tf, jnp.int32)
                    trial = t | bitv
                    cnt = count_ge(trial)
                    nt = t | (bitv & ge_mask(cnt))
                    tbuf[pl.ds(0, _L)] = lax.bitcast_convert_type(
                        nt, jnp.float32)
                    return cnt[0] == kf

                done = lax.cond(done, lambda _: done, live, 0)
                return (jnp.logical_or(done, bitv[0] == 1), bitv >> 1)

            lax.fori_loop(0, 30, wb, (done0, bitv0))
            tv = lax.bitcast_convert_type(tbuf[pl.ds(0, _L)], jnp.int32)

            def mb(v, _):
                xv = xbuf[r, pl.ds(v * _L, _L)]
                av = lax.bitcast_convert_type(xv, jnp.int32) & sign_off
                sel = jnp.where(av >= tv, xv, zero_f)
                xbuf[r, pl.ds(v * _L, _L)] = sel * alpha + xv * beta
                return 0

            lax.fori_loop(0, vpr, mb, 0, unroll=16)
            return 0

        lax.fori_loop(0, chunk, row_body, 0)
        pltpu.sync_copy(xbuf, out_hbm.at[pl.ds(row0, chunk)])
        return 0

    lax.fori_loop(0, rows_w // chunk, chunk_body, 0)


def _tc_body(scale_ref, x_ref, o_ref, *, k: int):
    xb = x_ref[...]
    a = jax.lax.bitcast_convert_type(jnp.abs(xb), jnp.int32)
    r = xb.shape[0]
    t = jnp.zeros((r, 1), jnp.int32)
    for b in range(30, -1, -1):
        trial = t + (1 << b)
        cnt = jnp.sum((a >= trial).astype(jnp.int32), axis=1, keepdims=True)
        t = jnp.where(cnt >= k, trial, t)
    sparse = jnp.where(a >= t, xb, jnp.zeros_like(xb))
    o_ref[...] = sparse * scale_ref[0] + xb * scale_ref[1]


def _sc_call(x2, scale, *, k, sc_rows, hw, chunk):
    rows_total = x2.shape[0]
    mesh = plsc.VectorSubcoreMesh(core_axis_name="c", subcore_axis_name="s")
    f = functools.partial(
        pl.kernel,
        mesh=mesh,
        out_type=jax.ShapeDtypeStruct((rows_total, hw), x2.dtype),
        scratch_types=[
            pltpu.VMEM((chunk, hw), jnp.float32),
            pltpu.VMEM((2 * _L,), jnp.float32),
            pltpu.VMEM((_L,), jnp.float32),
        ],
    )(functools.partial(_sc_body, k=k, rows=sc_rows, row_len=hw,
                        chunk=chunk))
    return f(x2, scale)


def _tc_call(x2, scale, *, k, hw, row_off):
    rows = x2.shape[0] - row_off
    blk = 256
    while rows % blk or (row_off and row_off % blk):
        blk //= 2
    off = row_off // blk if blk else 0
    return pl.pallas_call(
        functools.partial(_tc_body, k=k),
        grid=(rows // blk,),
        in_specs=[
            pl.BlockSpec(memory_space=pltpu.SMEM),
            pl.BlockSpec((blk, hw), lambda i: (i + off, 0)),
        ],
        out_specs=pl.BlockSpec((blk, hw), lambda i: (i + off, 0)),
        out_shape=jax.ShapeDtypeStruct((x2.shape[0], hw), x2.dtype),
    )(scale, x2)


def kernel(x, tau):
    n, c, h, w = x.shape
    hw = h * w
    k = max(int(0.1 * hw), 1)
    rows = n * c
    x2 = x.reshape(rows, hw)

    tau_f = jnp.asarray(tau, x.dtype)
    is_id = tau_f == jnp.asarray(1.0, x.dtype)
    alpha = jnp.where(is_id, jnp.asarray(1.0, x.dtype), tau_f)
    beta = jnp.where(is_id, jnp.asarray(0.0, x.dtype), 1.0 - tau_f)
    scale_sc = jnp.concatenate(
        [jnp.full((_L,), alpha, x.dtype), jnp.full((_L,), beta, x.dtype)])
    scale_tc = jnp.stack([alpha, beta])

    chunk = 16
    grain = _NC * _NS * chunk
    sc_rows = ((rows * 1) // 3) // grain * grain
    if sc_rows == 0 or rows % grain:
        sc_rows = rows - rows % grain
    tc_rows = rows - sc_rows

    if sc_rows and tc_rows:
        sc_out = _sc_call(x2[:sc_rows], scale_sc, k=k, sc_rows=sc_rows,
                          hw=hw, chunk=chunk)
        tc_out = _tc_call(x2[sc_rows:], scale_tc, k=k, hw=hw, row_off=0)
        out = jnp.concatenate([sc_out, tc_out], axis=0)
    elif sc_rows:
        out = _sc_call(x2, scale_sc, k=k, sc_rows=sc_rows, hw=hw,
                       chunk=chunk)
    else:
        out = _tc_call(x2, scale_tc, k=k, hw=hw, row_off=0)
    return out.reshape(n, c, h, w)
